# Initial kernel scaffold; baseline (speedup 1.0000x reference)
import jax
import jax.numpy as jnp
from jax.experimental import pallas as pl


def kernel(z_h, edge_index_h_h, edge_index_world, eW1, eb1, eW2, eb2, wW1, wb1, wW2, wb2, ewW1, ewb1, ewW2, ewb2, wwW1, wwb1, wwW2, wwb2, nW1, nb1, nW2, nb2):
    raise NotImplementedError("write your pallas kernel here")


# speedup vs baseline: 21.2770x; 21.2770x over previous
import functools

import jax
import jax.numpy as jnp
from jax import lax
from jax.experimental import pallas as pl
from jax.experimental.pallas import tpu as pltpu
from jax.experimental.pallas import tpu_sc as plsc

F = 13
MD = 128
HD = 256
N = 10000
NP = 10240
DUMMY = N
E_H = 320000
E_W = 10000
K = 128
NBUF = 4
NC, NS = 2, 16
NW = NC * NS
EHP = NW * K * 80
EWP = NW * K * 4
E_TOT = 2 * EHP + 2 * EWP
CH_W = E_TOT // NW // K
T = 2048
TN = 1280

@functools.lru_cache(maxsize=None)
def _get_mesh():
    return plsc.VectorSubcoreMesh(core_axis_name="c", subcore_axis_name="s",
                                  num_cores=NC, num_subcores=NS)


@functools.lru_cache(maxsize=None)
def _get_gather():
    @functools.partial(
        pl.kernel,
        out_type=jax.ShapeDtypeStruct((E_TOT, 16), jnp.float32),
        mesh=_get_mesh(),
        scratch_types=[
            pltpu.VMEM((CH_W, K), jnp.int32),
            pltpu.VMEM((NBUF, K, 16), jnp.float32),
            pltpu.SemaphoreType.DMA,
            pltpu.SemaphoreType.DMA,
        ],
        compiler_params=pltpu.CompilerParams(use_tc_tiling_on_sc=False),
    )
    def _gather_sc(z_hbm, idx_hbm, out_hbm, idx_v, rows_v, gsem, ssem):
        wid = lax.axis_index("s") * NC + lax.axis_index("c")
        cbase = wid * CH_W
        pltpu.sync_copy(idx_hbm.at[pl.ds(cbase, CH_W)], idx_v)

        def step(o, _):
            gd = [
                pltpu.async_copy(z_hbm.at[idx_v.at[o * NBUF + b]],
                                 rows_v.at[b], gsem)
                for b in range(NBUF)
            ]
            for b in range(NBUF):
                gd[b].wait()
            sd = [
                pltpu.async_copy(
                    rows_v.at[b],
                    out_hbm.at[pl.ds((cbase + o * NBUF + b) * K, K)],
                    ssem,
                )
                for b in range(NBUF)
            ]
            for b in range(NBUF):
                sd[b].wait()
            return ()

        lax.fori_loop(0, CH_W // NBUF, step, ())

    return _gather_sc


def _edge_body(zs_ref, zt_ref, W1_ref, b1_ref, W2_ref, b2_ref, wv_ref, wb_ref,
               out_ref):
    zs = zs_ref[...]
    zt = zt_ref[...]
    s3, s4, s5 = zs[:, 3:4], zs[:, 4:5], zs[:, 5:6]
    t3, t4, t5 = zt[:, 3:4], zt[:, 4:5], zt[:, 5:6]
    d0 = zs[:, 0:1] - zt[:, 0:1]
    d1 = zs[:, 1:2] - zt[:, 1:2]
    d2 = zs[:, 2:3] - zt[:, 2:3]
    dist = d0 * d0 + d1 * d1 + d2 * d2
    c0 = s4 * t5 - s5 * t4
    c1 = s5 * t3 - s3 * t5
    c2 = s3 * t4 - s4 * t3
    ac = jnp.sqrt(c0 * c0 + c1 * c1 + c2 * c2)
    inp = jnp.concatenate(
        [zs[:, 0:F], zt[:, 0:F], d0, d1, d2, dist, c0, c1, c2, ac], axis=1)
    h = jnp.tanh(
        lax.dot_general(inp, W1_ref[...], (((1,), (0,)), ((), ())),
                        preferred_element_type=jnp.float32) + b1_ref[...])
    m = lax.dot_general(h[:, 0:HD], W2_ref[...], (((1,), (0,)), ((), ())),
                        preferred_element_type=jnp.float32) + b2_ref[...]
    g = jnp.sum(h[:, HD:2 * HD] * wv_ref[...], axis=1, keepdims=True) + wb_ref[...]
    out_ref[...] = m * (1.0 / (1.0 + jnp.exp(-g)))


def _im_shift(off, i):
    return (off + i, 0)


def _im_zero(i):
    return (0, 0)


def _edge_call(gathered, W1, b1, W2, b2, wv, wb, EP, off_s, off_t):
    G = EP // T
    return pl.pallas_call(
        _edge_body,
        grid=(G,),
        in_specs=[
            pl.BlockSpec((T, 16), functools.partial(_im_shift, off_s // T)),
            pl.BlockSpec((T, 16), functools.partial(_im_shift, off_t // T)),
            pl.BlockSpec((2 * F + 8, 2 * HD), _im_zero),
            pl.BlockSpec((1, 2 * HD), _im_zero),
            pl.BlockSpec((HD, MD), _im_zero),
            pl.BlockSpec((1, MD), _im_zero),
            pl.BlockSpec((1, HD), _im_zero),
            pl.BlockSpec((1, 1), _im_zero),
        ],
        out_specs=pl.BlockSpec((T, MD), lambda i: (i, 0)),
        out_shape=jax.ShapeDtypeStruct((EP, MD), jnp.float32),
        compiler_params=pltpu.CompilerParams(
            dimension_semantics=("arbitrary",)),
    )(gathered, gathered, W1, b1, W2, b2, wv, wb)


CH_H = EHP // NW // K
CH_WD = EWP // NW // K
HALF = NP // 2
ACC_R = HALF + K
ZCH = ACC_R // K
WCH = HALF // K


@functools.lru_cache(maxsize=None)
def _get_scatter():
    @functools.partial(
        pl.kernel,
        out_type=(jax.ShapeDtypeStruct((NC, NP, MD), jnp.float32),
                  jax.ShapeDtypeStruct((NC, NP, MD), jnp.float32)),
        mesh=_get_mesh(),
        scratch_types=[
            pltpu.VMEM((CH_H, K), jnp.int32),
            pltpu.VMEM((NBUF, K), jnp.int32),
            pltpu.VMEM((NBUF, K, MD), jnp.float32),
            pltpu.VMEM_SHARED((ACC_R, MD), jnp.float32),
            pltpu.SemaphoreType.DMA,
        ],
    )
    def _scatter_sc(msgsh_hbm, tgth_hbm, msgsw_hbm, tgtw_hbm, outh_hbm,
                    outw_hbm, idx_v, ridx_v, buf_v, acc_sh, sem):
        cid = lax.axis_index("c")
        sid = lax.axis_index("s")
        wid = sid * NC + cid
        zero16 = jnp.zeros((16,), jnp.float32)

        def zloop(j, _):
            buf_v[0, j // 8, pl.ds((j % 8) * 16, 16)] = zero16
            return ()

        def zero_acc():
            for i in range(3):
                c = sid + i * NS
                if i * NS + NS <= ZCH:
                    pltpu.sync_copy(buf_v.at[0], acc_sh.at[pl.ds(c * K, K)])
                else:
                    @pl.when(c < ZCH)
                    def _():
                        pltpu.sync_copy(buf_v.at[0],
                                        acc_sh.at[pl.ds(c * K, K)])

        def accumulate(msgs_hbm, CH, lo):
            cbase = wid * CH

            def step(o, _):
                gd = [
                    pltpu.async_copy(
                        msgs_hbm.at[pl.ds((cbase + o * NBUF + b) * K, K)],
                        buf_v.at[b], sem)
                    for b in range(NBUF)
                ]
                for b in range(NBUF):
                    for g in range(K // 16):
                        r = idx_v[o * NBUF + b, pl.ds(g * 16, 16)] - lo
                        ok = r.astype(jnp.uint32) < jnp.uint32(HALF)
                        ridx_v[b, pl.ds(g * 16, 16)] = jnp.where(
                            ok, r, jnp.int32(HALF))
                    gd[b].wait()
                    pltpu.sync_copy(buf_v.at[b], acc_sh.at[ridx_v.at[b]],
                                    add=True)
                return ()

            lax.fori_loop(0, CH // NBUF, step, ())

        def writeback(out_hbm, lo):
            for i in range(3):
                c = sid + i * NS
                b = 1 + i % (NBUF - 1)
                if not (i * NS + NS <= WCH):
                    @pl.when(c < WCH)
                    def _():
                        pltpu.sync_copy(acc_sh.at[pl.ds(c * K, K)],
                                        buf_v.at[b])
                        pltpu.sync_copy(
                            buf_v.at[b],
                            out_hbm.at[cid, pl.ds(lo + c * K, K)])
                else:
                    pltpu.sync_copy(acc_sh.at[pl.ds(c * K, K)], buf_v.at[b])
                    pltpu.sync_copy(buf_v.at[b],
                                    out_hbm.at[cid, pl.ds(lo + c * K, K)])

        for branch in range(2):
            msgs_hbm = (msgsh_hbm, msgsw_hbm)[branch]
            tgt_hbm = (tgth_hbm, tgtw_hbm)[branch]
            out_hbm = (outh_hbm, outw_hbm)[branch]
            CH = (CH_H, CH_WD)[branch]
            pltpu.sync_copy(tgt_hbm.at[pl.ds(wid * CH, CH)],
                            idx_v.at[pl.ds(0, CH)])
            for half in range(2):
                lo = half * HALF
                lax.fori_loop(0, K * MD // 16, zloop, ())
                zero_acc()
                plsc.subcore_barrier()
                accumulate(msgs_hbm, CH, lo)
                plsc.subcore_barrier()
                writeback(out_hbm, lo)
                plsc.subcore_barrier()

    return _scatter_sc


def _node_body(z_ref, ah0_ref, ah1_ref, aw0_ref, aw1_ref, A_ref, Bh_ref,
               Bw_ref, b1_ref, W2_ref, b2_ref, out_ref):
    mh = ah0_ref[0] + ah1_ref[0]
    mw = aw0_ref[0] + aw1_ref[0]
    acc = lax.dot_general(z_ref[...], A_ref[...], (((1,), (0,)), ((), ())),
                          preferred_element_type=jnp.float32)
    acc += lax.dot_general(mh, Bh_ref[...], (((1,), (0,)), ((), ())),
                           preferred_element_type=jnp.float32)
    acc += lax.dot_general(mw, Bw_ref[...], (((1,), (0,)), ((), ())),
                           preferred_element_type=jnp.float32)
    h = jnp.tanh(acc + b1_ref[...])
    out_ref[...] = lax.dot_general(h, W2_ref[...], (((1,), (0,)), ((), ())),
                                   preferred_element_type=jnp.float32) + b2_ref[...]


def _node_call(z16, aggh, aggw, A, Bh, Bw, b1, W2, b2):
    G = NP // TN
    return pl.pallas_call(
        _node_body,
        grid=(G,),
        in_specs=[
            pl.BlockSpec((TN, 16), lambda i: (i, 0)),
            pl.BlockSpec((1, TN, MD), lambda i: (0, i, 0)),
            pl.BlockSpec((1, TN, MD), lambda i: (1, i, 0)),
            pl.BlockSpec((1, TN, MD), lambda i: (0, i, 0)),
            pl.BlockSpec((1, TN, MD), lambda i: (1, i, 0)),
            pl.BlockSpec((16, HD), _im_zero),
            pl.BlockSpec((MD, HD), _im_zero),
            pl.BlockSpec((MD, HD), _im_zero),
            pl.BlockSpec((1, HD), _im_zero),
            pl.BlockSpec((HD, 16), _im_zero),
            pl.BlockSpec((1, 16), _im_zero),
        ],
        out_specs=pl.BlockSpec((TN, 16), lambda i: (i, 0)),
        out_shape=jax.ShapeDtypeStruct((NP, 16), jnp.float32),
        compiler_params=pltpu.CompilerParams(
            dimension_semantics=("arbitrary",)),
    )(z16, aggh, aggh, aggw, aggw, A, Bh, Bw, b1, W2, b2)


def kernel(z_h, edge_index_h_h, edge_index_world, eW1, eb1, eW2, eb2, wW1,
           wb1, wW2, wb2, ewW1, ewb1, ewW2, ewb2, wwW1, wwb1, wwW2, wwb2,
           nW1, nb1, nW2, nb2):
    z16 = jnp.pad(z_h[0], ((0, NP - N), (0, 16 - F)))
    srcH = jnp.pad(edge_index_h_h[0], (0, EHP - E_H))
    tgtH = jnp.pad(edge_index_h_h[1], (0, EHP - E_H), constant_values=DUMMY)
    srcW = jnp.pad(edge_index_world[0], (0, EWP - E_W))
    tgtW = jnp.pad(edge_index_world[1], (0, EWP - E_W), constant_values=DUMMY)
    idx2d = jnp.concatenate([srcH, tgtH, srcW, tgtW]).reshape(E_TOT // K, K)

    gathered = _get_gather()(z16, idx2d)

    W1h = jnp.concatenate([eW1, wW1], axis=1)
    b1h = jnp.concatenate([eb1, wb1])[None, :]
    W1w = jnp.concatenate([ewW1, wwW1], axis=1)
    b1w = jnp.concatenate([ewb1, wwb1])[None, :]
    msgs_h = _edge_call(gathered, W1h, b1h, eW2, eb2[None, :],
                        wW2.reshape(1, HD), wb2.reshape(1, 1),
                        EHP, 0, EHP)
    msgs_w = _edge_call(gathered, W1w, b1w, ewW2, ewb2[None, :],
                        wwW2.reshape(1, HD), wwb2.reshape(1, 1),
                        EWP, 2 * EHP, 2 * EHP + EWP)

    aggh, aggw = _get_scatter()(msgs_h, tgtH.reshape(EHP // K, K),
                                msgs_w, tgtW.reshape(EWP // K, K))

    A = jnp.pad(nW1[0:F], ((0, 16 - F), (0, 0)))
    Bh = nW1[F:F + MD]
    Bw = nW1[F + MD:F + 2 * MD]
    W2n = jnp.pad(nW2, ((0, 0), (0, 16 - F)))
    b2n = jnp.pad(nb2, (0, 16 - F))[None, :]
    outn = _node_call(z16, aggh, aggw, A, Bh, Bw, nb1[None, :], W2n, b2n)
    return outn[:N, :F][None]
